# Initial kernel scaffold; baseline (speedup 1.0000x reference)
import jax
import jax.numpy as jnp
from jax.experimental import pallas as pl


def kernel(main_feat, other_feat, fix_feat, Wq, bq, Wk, bk):
    raise NotImplementedError("write your pallas kernel here")


# speedup vs baseline: 1.6236x; 1.6236x over previous
import math

import jax
import jax.numpy as jnp
from jax.experimental import pallas as pl
from jax.experimental.pallas import tpu as pltpu

QDIM = 256
MID_D = 256
N_MAIN = 10000
N_OTHER = 1024
BM = 1000
SCALE = 1.0 / math.sqrt(MID_D)


def _pre_kernel(fix_ref, other_ref, wk_ref, bk_ref, om_ref, k_ref):
    fix = fix_ref[...]
    g = jax.lax.dot_general(fix, fix, (((0,), (0,)), ((), ())))
    ffraw = jnp.sqrt(g)
    colsum = jnp.sum(ffraw, axis=0)[:, None]
    other = other_ref[...]
    om_ref[...] = jnp.dot(ffraw, other / colsum)
    k_ref[...] = jnp.dot(other, wk_ref[...].T) + bk_ref[...]


def _attn_kernel(main_ref, wq_ref, bq_ref, k_ref, om_ref, out_ref):
    q = jnp.dot(main_ref[...], wq_ref[...].T) + bq_ref[...]
    a = jax.lax.dot_general(q, k_ref[...], (((1,), (1,)), ((), ()))) * SCALE
    m = jnp.max(a, axis=1, keepdims=True)
    p = jnp.exp(a - m)
    o = jnp.dot(p, om_ref[...])
    out_ref[...] = o / jnp.sum(p, axis=1, keepdims=True)


def kernel(main_feat, other_feat, fix_feat, Wq, bq, Wk, bk):
    bq2 = bq.reshape(1, MID_D)
    bk2 = bk.reshape(1, MID_D)

    om, K = pl.pallas_call(
        _pre_kernel,
        out_shape=(
            jax.ShapeDtypeStruct((N_OTHER, MID_D), jnp.float32),
            jax.ShapeDtypeStruct((N_OTHER, MID_D), jnp.float32),
        ),
    )(fix_feat, other_feat, Wk, bk2)

    O = pl.pallas_call(
        _attn_kernel,
        grid=(N_MAIN // BM,),
        in_specs=[
            pl.BlockSpec((BM, QDIM), lambda i: (i, 0)),
            pl.BlockSpec((MID_D, QDIM), lambda i: (0, 0)),
            pl.BlockSpec((1, MID_D), lambda i: (0, 0)),
            pl.BlockSpec((N_OTHER, MID_D), lambda i: (0, 0)),
            pl.BlockSpec((N_OTHER, MID_D), lambda i: (0, 0)),
        ],
        out_specs=pl.BlockSpec((BM, MID_D), lambda i: (i, 0)),
        out_shape=jax.ShapeDtypeStruct((N_MAIN, MID_D), jnp.float32),
        compiler_params=pltpu.CompilerParams(
            dimension_semantics=("arbitrary",),
        ),
    )(main_feat, Wq, bq2, K, om)
    return O


# speedup vs baseline: 1.6634x; 1.0245x over previous
import math

import jax
import jax.numpy as jnp
from jax.experimental import pallas as pl
from jax.experimental.pallas import tpu as pltpu

QDIM = 256
MID_D = 256
N_MAIN = 10000
N_OTHER = 1024
BM = 2000
SCALE = 1.0 / math.sqrt(MID_D)


def _bf(x):
    return x.astype(jnp.bfloat16)


def _dot(a, b, dims):
    return jax.lax.dot_general(_bf(a), _bf(b), (dims, ((), ())),
                               preferred_element_type=jnp.float32)


def _pre_kernel(fix_ref, other_ref, wk_ref, bk_ref, om_ref, k_ref):
    fix = fix_ref[...]
    g = _dot(fix, fix, ((0,), (0,)))
    ffraw = jnp.sqrt(g)
    colsum = jnp.sum(ffraw, axis=0)[:, None]
    other = other_ref[...]
    om_ref[...] = _dot(ffraw, other / colsum, ((1,), (0,)))
    k_ref[...] = _dot(other, wk_ref[...], ((1,), (1,))) + bk_ref[...]


def _attn_kernel(main_ref, wq_ref, bq_ref, k_ref, om_ref, out_ref):
    q = _dot(main_ref[...], wq_ref[...], ((1,), (1,))) + bq_ref[...]
    a = _dot(q, k_ref[...], ((1,), (1,))) * SCALE
    m = jnp.max(a, axis=1, keepdims=True)
    p = jnp.exp(a - m)
    o = _dot(p, om_ref[...], ((1,), (0,)))
    out_ref[...] = o / jnp.sum(p, axis=1, keepdims=True)


def kernel(main_feat, other_feat, fix_feat, Wq, bq, Wk, bk):
    bq2 = bq.reshape(1, MID_D)
    bk2 = bk.reshape(1, MID_D)

    om, K = pl.pallas_call(
        _pre_kernel,
        out_shape=(
            jax.ShapeDtypeStruct((N_OTHER, MID_D), jnp.float32),
            jax.ShapeDtypeStruct((N_OTHER, MID_D), jnp.float32),
        ),
    )(fix_feat, other_feat, Wk, bk2)

    O = pl.pallas_call(
        _attn_kernel,
        grid=(N_MAIN // BM,),
        in_specs=[
            pl.BlockSpec((BM, QDIM), lambda i: (i, 0)),
            pl.BlockSpec((MID_D, QDIM), lambda i: (0, 0)),
            pl.BlockSpec((1, MID_D), lambda i: (0, 0)),
            pl.BlockSpec((N_OTHER, MID_D), lambda i: (0, 0)),
            pl.BlockSpec((N_OTHER, MID_D), lambda i: (0, 0)),
        ],
        out_specs=pl.BlockSpec((BM, MID_D), lambda i: (i, 0)),
        out_shape=jax.ShapeDtypeStruct((N_MAIN, MID_D), jnp.float32),
        compiler_params=pltpu.CompilerParams(
            dimension_semantics=("arbitrary",),
        ),
    )(main_feat, Wq, bq2, K, om)
    return O
